# speedup vs baseline: 5.8842x; 5.8842x over previous
import jax
import jax.numpy as jnp
from jax import lax
from jax.experimental import pallas as pl
from jax.experimental.pallas import tpu as pltpu
from jax.experimental.pallas import tpu_sc as plsc

EMBED = 128
KC = 4
ASIZE = EMBED // KC


def _sc_gather(node_features, neighbors):
    B = neighbors.shape[0]
    D = node_features.shape[1]
    info = plsc.get_sparse_core_info()
    NW = info.num_cores * info.num_subcores
    b_per_w = B // NW
    CH = 200
    NB = 2
    n_super = b_per_w // (CH * NB)
    assert b_per_w % (CH * NB) == 0

    mesh = plsc.VectorSubcoreMesh(core_axis_name="c", subcore_axis_name="s")

    def body(nf_hbm, idx_hbm, out_hbm, idx_v, rows_v, gsem, ssem):
        cid = lax.axis_index("c")
        sid = lax.axis_index("s")
        wid = sid * info.num_cores + cid
        base = wid * b_per_w
        pltpu.sync_copy(idx_hbm.at[pl.ds(base, b_per_w)], idx_v)

        def super_step(io, carry):
            c0 = io * NB
            handles = [
                pltpu.async_copy(
                    nf_hbm.at[idx_v.at[pl.ds((c0 + b) * CH, CH)]],
                    rows_v.at[b],
                    gsem,
                )
                for b in range(NB)
            ]
            for b in range(NB):
                handles[b].wait()
                pltpu.async_copy(
                    rows_v.at[b],
                    out_hbm.at[pl.ds(base + (c0 + b) * CH, CH)],
                    ssem,
                ).wait()
            return carry

        lax.fori_loop(0, n_super, super_step, 0)

    f = pl.kernel(
        body,
        out_type=jax.ShapeDtypeStruct((B, D), jnp.float32),
        mesh=mesh,
        scratch_types=[
            pltpu.VMEM((b_per_w,), jnp.int32),
            pltpu.VMEM((NB, CH, D), jnp.float32),
            pltpu.SemaphoreType.DMA,
            pltpu.SemaphoreType.DMA,
        ],
    )
    return f(node_features, neighbors)


def _routing(node_features, z_flat, num_node, num_neib):
    NBLK = 80
    RBLK = NBLK * num_neib

    def kern(nf_ref, z_ref, out_ref):
        d_i = lax.broadcasted_iota(jnp.int32, (EMBED, KC), 0)
        k_i = lax.broadcasted_iota(jnp.int32, (EMBED, KC), 1)
        S = (d_i // ASIZE == k_i).astype(jnp.float32)
        k_t = lax.broadcasted_iota(jnp.int32, (KC, EMBED), 0)
        d_t = lax.broadcasted_iota(jnp.int32, (KC, EMBED), 1)
        St = (d_t // ASIZE == k_t).astype(jnp.float32)

        def seg_matmul(x):
            return lax.dot_general(
                x, S, (((1,), (0,)), ((), ())),
                preferred_element_type=jnp.float32)

        def expand(c):
            return lax.dot_general(
                c, St, (((1,), (0,)), ((), ())),
                preferred_element_type=jnp.float32)

        def normalize(x):
            s2 = seg_matmul(x * x)
            inv = 1.0 / jnp.maximum(jnp.sqrt(s2), 1e-12)
            return x * expand(inv)

        h = normalize(nf_ref[...])
        z = normalize(z_ref[...])

        def seg_sum_neib(x):
            return jnp.sum(x.reshape(NBLK, num_neib, EMBED), axis=1)

        u = normalize((1.0 / KC) * seg_sum_neib(z) + h)
        for it in (1, 2):
            ub = jnp.broadcast_to(
                u[:, None, :], (NBLK, num_neib, EMBED)
            ).reshape(RBLK, EMBED)
            logits = seg_matmul(z * ub)
            m = jnp.max(logits, axis=-1, keepdims=True)
            e = jnp.exp(logits - m)
            w = e / jnp.sum(e, axis=-1, keepdims=True)
            u = seg_sum_neib(z * expand(w)) + h
            if it != 2:
                u = normalize(u)
        out_ref[...] = u

    num_blocks = num_node // NBLK
    return pl.pallas_call(
        kern,
        grid=(num_blocks,),
        in_specs=[
            pl.BlockSpec((NBLK, EMBED), lambda i: (i, 0)),
            pl.BlockSpec((RBLK, EMBED), lambda i: (i, 0)),
        ],
        out_specs=pl.BlockSpec((NBLK, EMBED), lambda i: (i, 0)),
        out_shape=jax.ShapeDtypeStruct((num_node, EMBED), jnp.float32),
    )(node_features, z_flat)


def kernel(node_features, neighbors, edgenode_iter):
    num_node = node_features.shape[0]
    num_neib = neighbors.shape[0] // num_node
    z_flat = _sc_gather(node_features, neighbors.astype(jnp.int32))
    return _routing(node_features, z_flat, num_node, num_neib)


# speedup vs baseline: 8.5136x; 1.4469x over previous
import jax
import jax.numpy as jnp
from jax import lax
from jax.experimental import pallas as pl
from jax.experimental.pallas import tpu as pltpu
from jax.experimental.pallas import tpu_sc as plsc

EMBED = 128
KC = 4
ASIZE = EMBED // KC


def _sc_gather(node_features, neighbors):
    B = neighbors.shape[0]
    D = node_features.shape[1]
    info = plsc.get_sparse_core_info()
    NW = info.num_cores * info.num_subcores
    b_per_w = B // NW
    CH = 200
    NB = 2
    n_super = b_per_w // (CH * NB)
    assert b_per_w % (CH * NB) == 0

    mesh = plsc.VectorSubcoreMesh(core_axis_name="c", subcore_axis_name="s")

    def body(nf_hbm, idx_hbm, out_hbm, idx_v, rows_v, gsem, ssem):
        cid = lax.axis_index("c")
        sid = lax.axis_index("s")
        wid = sid * info.num_cores + cid
        base = wid * b_per_w
        pltpu.sync_copy(idx_hbm.at[pl.ds(base, b_per_w)], idx_v)

        def super_step(io, carry):
            c0 = io * NB
            handles = [
                pltpu.async_copy(
                    nf_hbm.at[idx_v.at[pl.ds((c0 + b) * CH, CH)]],
                    rows_v.at[b],
                    gsem,
                )
                for b in range(NB)
            ]
            for b in range(NB):
                handles[b].wait()
                pltpu.async_copy(
                    rows_v.at[b],
                    out_hbm.at[pl.ds(base + (c0 + b) * CH, CH)],
                    ssem,
                ).wait()
            return carry

        lax.fori_loop(0, n_super, super_step, 0)

    f = pl.kernel(
        body,
        out_type=jax.ShapeDtypeStruct((B, D), jnp.float32),
        mesh=mesh,
        scratch_types=[
            pltpu.VMEM((b_per_w,), jnp.int32),
            pltpu.VMEM((NB, CH, D), jnp.float32),
            pltpu.SemaphoreType.DMA,
            pltpu.SemaphoreType.DMA,
        ],
    )
    return f(node_features, neighbors)


def _routing(node_features, z_flat, num_node, num_neib):
    NBLK = 200
    RBLK = NBLK * num_neib

    def kern(nf_ref, z_ref, out_ref):
        d_i = lax.broadcasted_iota(jnp.int32, (EMBED, EMBED), 0)
        d_j = lax.broadcasted_iota(jnp.int32, (EMBED, EMBED), 1)
        P = (d_i // ASIZE == d_j // ASIZE).astype(jnp.float32)

        def seg_sums(x):
            return lax.dot_general(
                x, P, (((1,), (0,)), ((), ())),
                preferred_element_type=jnp.float32)

        def normalize(x):
            s2 = seg_sums(x * x)
            return x * (1.0 / jnp.maximum(jnp.sqrt(s2), 1e-12))

        h = normalize(nf_ref[...])
        z = normalize(z_ref[...])

        def seg_sum_neib(x):
            return jnp.sum(x.reshape(NBLK, num_neib, EMBED), axis=1)

        u = normalize((1.0 / KC) * seg_sum_neib(z) + h)
        for it in (1, 2):
            ub = jnp.broadcast_to(
                u[:, None, :], (NBLK, num_neib, EMBED)
            ).reshape(RBLK, EMBED)
            logits = seg_sums(z * ub)
            e = jnp.exp(logits)
            denom = jnp.sum(e, axis=-1, keepdims=True) * (1.0 / ASIZE)
            w = e / denom
            u = seg_sum_neib(z * w) + h
            if it != 2:
                u = normalize(u)
        out_ref[...] = u

    num_blocks = num_node // NBLK
    return pl.pallas_call(
        kern,
        grid=(num_blocks,),
        in_specs=[
            pl.BlockSpec((NBLK, EMBED), lambda i: (i, 0)),
            pl.BlockSpec((RBLK, EMBED), lambda i: (i, 0)),
        ],
        out_specs=pl.BlockSpec((NBLK, EMBED), lambda i: (i, 0)),
        out_shape=jax.ShapeDtypeStruct((num_node, EMBED), jnp.float32),
    )(node_features, z_flat)


def kernel(node_features, neighbors, edgenode_iter):
    num_node = node_features.shape[0]
    num_neib = neighbors.shape[0] // num_node
    z_flat = _sc_gather(node_features, neighbors.astype(jnp.int32))
    return _routing(node_features, z_flat, num_node, num_neib)


# speedup vs baseline: 9.4683x; 1.1121x over previous
import jax
import jax.numpy as jnp
from jax import lax
from jax.experimental import pallas as pl
from jax.experimental.pallas import tpu as pltpu
from jax.experimental.pallas import tpu_sc as plsc

EMBED = 128
KC = 4
ASIZE = EMBED // KC


def _sc_gather(node_features, neighbors):
    B = neighbors.shape[0]
    D = node_features.shape[1]
    info = plsc.get_sparse_core_info()
    NW = info.num_cores * info.num_subcores
    b_per_w = B // NW
    CH = 200
    NB = 2
    n_super = b_per_w // (CH * NB)
    assert b_per_w % (CH * NB) == 0

    mesh = plsc.VectorSubcoreMesh(core_axis_name="c", subcore_axis_name="s")

    def body(nf_hbm, idx_hbm, out_hbm, idx_v, rows_v, gsem, ssem):
        cid = lax.axis_index("c")
        sid = lax.axis_index("s")
        wid = sid * info.num_cores + cid
        base = wid * b_per_w
        pltpu.sync_copy(idx_hbm.at[pl.ds(base, b_per_w)], idx_v)

        def super_step(io, carry):
            c0 = io * NB
            handles = [
                pltpu.async_copy(
                    nf_hbm.at[idx_v.at[pl.ds((c0 + b) * CH, CH)]],
                    rows_v.at[b],
                    gsem,
                )
                for b in range(NB)
            ]
            for b in range(NB):
                handles[b].wait()
                pltpu.async_copy(
                    rows_v.at[b],
                    out_hbm.at[pl.ds(base + (c0 + b) * CH, CH)],
                    ssem,
                ).wait()
            return carry

        lax.fori_loop(0, n_super, super_step, 0)

    f = pl.kernel(
        body,
        out_type=jax.ShapeDtypeStruct((B, D), jnp.float32),
        mesh=mesh,
        scratch_types=[
            pltpu.VMEM((b_per_w,), jnp.int32),
            pltpu.VMEM((NB, CH, D), jnp.float32),
            pltpu.SemaphoreType.DMA,
            pltpu.SemaphoreType.DMA,
        ],
    )
    return f(node_features, neighbors)


def _routing(node_features, z_flat, num_node, num_neib):
    NBLK = 200
    RBLK = NBLK * num_neib

    def kern(nf_ref, z_ref, out_ref):
        d_i = lax.broadcasted_iota(jnp.int32, (EMBED, EMBED), 0)
        d_j = lax.broadcasted_iota(jnp.int32, (EMBED, EMBED), 1)
        P = (d_i // ASIZE == d_j // ASIZE).astype(jnp.float32)
        Pall = jnp.full((EMBED, EMBED), 1.0 / ASIZE, dtype=jnp.float32)

        def seg_sums(x):
            return lax.dot_general(
                x, P, (((1,), (0,)), ((), ())),
                preferred_element_type=jnp.float32)

        def normalize(x):
            s2 = seg_sums(x * x)
            return x * jnp.minimum(lax.rsqrt(s2), 1e12)

        h = normalize(nf_ref[...])
        z = normalize(z_ref[...])

        def seg_sum_neib(x):
            return jnp.sum(x.reshape(NBLK, num_neib, EMBED), axis=1)

        u = normalize((1.0 / KC) * seg_sum_neib(z) + h)
        for it in (1, 2):
            ub = jnp.broadcast_to(
                u[:, None, :], (NBLK, num_neib, EMBED)
            ).reshape(RBLK, EMBED)
            logits = seg_sums(z * ub)
            e = jnp.exp(logits)
            denom = lax.dot_general(
                e, Pall, (((1,), (0,)), ((), ())),
                preferred_element_type=jnp.float32)
            w = e / denom
            u = seg_sum_neib(z * w) + h
            if it != 2:
                u = normalize(u)
        out_ref[...] = u

    num_blocks = num_node // NBLK
    return pl.pallas_call(
        kern,
        grid=(num_blocks,),
        in_specs=[
            pl.BlockSpec((NBLK, EMBED), lambda i: (i, 0)),
            pl.BlockSpec((RBLK, EMBED), lambda i: (i, 0)),
        ],
        out_specs=pl.BlockSpec((NBLK, EMBED), lambda i: (i, 0)),
        out_shape=jax.ShapeDtypeStruct((num_node, EMBED), jnp.float32),
    )(node_features, z_flat)


def kernel(node_features, neighbors, edgenode_iter):
    num_node = node_features.shape[0]
    num_neib = neighbors.shape[0] // num_node
    z_flat = _sc_gather(node_features, neighbors.astype(jnp.int32))
    return _routing(node_features, z_flat, num_node, num_neib)


# speedup vs baseline: 10.7160x; 1.1318x over previous
import jax
import jax.numpy as jnp
from jax import lax
from jax.experimental import pallas as pl
from jax.experimental.pallas import tpu as pltpu
from jax.experimental.pallas import tpu_sc as plsc

EMBED = 128
KC = 4
ASIZE = EMBED // KC


def _sc_gather(node_features, neighbors):
    B = neighbors.shape[0]
    D = node_features.shape[1]
    info = plsc.get_sparse_core_info()
    NW = info.num_cores * info.num_subcores
    b_per_w = B // NW
    CH = 200
    NB = 2
    n_ch = b_per_w // CH
    assert b_per_w % CH == 0
    n_super = n_ch // NB
    n_tail = n_ch - n_super * NB

    mesh = plsc.VectorSubcoreMesh(core_axis_name="c", subcore_axis_name="s")

    def body(nf_hbm, idx_hbm, out_hbm, idx_v, rows_v, gsem, ssem):
        cid = lax.axis_index("c")
        sid = lax.axis_index("s")
        wid = sid * info.num_cores + cid
        base = wid * b_per_w
        pltpu.sync_copy(idx_hbm.at[pl.ds(base, b_per_w)], idx_v)

        def super_step(io, carry):
            c0 = io * NB
            handles = [
                pltpu.async_copy(
                    nf_hbm.at[idx_v.at[pl.ds((c0 + b) * CH, CH)]],
                    rows_v.at[b],
                    gsem,
                )
                for b in range(NB)
            ]
            for b in range(NB):
                handles[b].wait()
                pltpu.async_copy(
                    rows_v.at[b],
                    out_hbm.at[pl.ds(base + (c0 + b) * CH, CH)],
                    ssem,
                ).wait()
            return carry

        lax.fori_loop(0, n_super, super_step, 0)
        for t in range(n_tail):
            c = n_super * NB + t
            pltpu.async_copy(
                nf_hbm.at[idx_v.at[pl.ds(c * CH, CH)]], rows_v.at[0], gsem
            ).wait()
            pltpu.async_copy(
                rows_v.at[0], out_hbm.at[pl.ds(base + c * CH, CH)], ssem
            ).wait()

    f = pl.kernel(
        body,
        out_type=jax.ShapeDtypeStruct((B, D), jnp.float32),
        mesh=mesh,
        scratch_types=[
            pltpu.VMEM((b_per_w,), jnp.int32),
            pltpu.VMEM((NB, CH, D), jnp.float32),
            pltpu.SemaphoreType.DMA,
            pltpu.SemaphoreType.DMA,
        ],
    )
    return f(node_features, neighbors)


def _routing(node_features, z_flat, num_node, num_neib):
    NBLK = 200
    RBLK = NBLK * num_neib

    def kern(nf_ref, z_ref, out_ref):
        d_i = lax.broadcasted_iota(jnp.int32, (EMBED, EMBED), 0)
        d_j = lax.broadcasted_iota(jnp.int32, (EMBED, EMBED), 1)
        P = (d_i // ASIZE == d_j // ASIZE).astype(jnp.float32)
        Pall = jnp.full((EMBED, EMBED), 1.0 / ASIZE, dtype=jnp.float32)

        def seg_sums(x):
            return lax.dot_general(
                x, P, (((1,), (0,)), ((), ())),
                preferred_element_type=jnp.float32)

        def normalize(x):
            s2 = seg_sums(x * x)
            return x * jnp.minimum(lax.rsqrt(s2), 1e12)

        h = normalize(nf_ref[...])
        z = normalize(z_ref[...])

        def seg_sum_neib(x):
            return jnp.sum(x.reshape(NBLK, num_neib, EMBED), axis=1)

        u = normalize((1.0 / KC) * seg_sum_neib(z) + h)
        for it in (1, 2):
            ub = jnp.broadcast_to(
                u[:, None, :], (NBLK, num_neib, EMBED)
            ).reshape(RBLK, EMBED)
            logits = seg_sums(z * ub)
            e = jnp.exp(logits)
            denom = lax.dot_general(
                e, Pall, (((1,), (0,)), ((), ())),
                preferred_element_type=jnp.float32)
            w = e / denom
            u = seg_sum_neib(z * w) + h
            if it != 2:
                u = normalize(u)
        out_ref[...] = u

    num_blocks = num_node // NBLK
    return pl.pallas_call(
        kern,
        grid=(num_blocks,),
        in_specs=[
            pl.BlockSpec((NBLK, EMBED), lambda i: (i, 0)),
            pl.BlockSpec((RBLK, EMBED), lambda i: (i, 0)),
        ],
        out_specs=pl.BlockSpec((NBLK, EMBED), lambda i: (i, 0)),
        out_shape=jax.ShapeDtypeStruct((num_node, EMBED), jnp.float32),
    )(node_features, z_flat)


def kernel(node_features, neighbors, edgenode_iter):
    num_node = node_features.shape[0]
    num_neib = neighbors.shape[0] // num_node
    nb = neighbors.astype(jnp.int32)
    half = num_node // 2
    e_half = half * num_neib
    z0 = _sc_gather(node_features, nb[:e_half])
    z1 = _sc_gather(node_features, nb[e_half:])
    u0 = _routing(node_features[:half], z0, half, num_neib)
    u1 = _routing(node_features[half:], z1, num_node - half, num_neib)
    return jnp.concatenate([u0, u1], axis=0)


# speedup vs baseline: 10.8451x; 1.0121x over previous
import jax
import jax.numpy as jnp
from jax import lax
from jax.experimental import pallas as pl
from jax.experimental.pallas import tpu as pltpu
from jax.experimental.pallas import tpu_sc as plsc

EMBED = 128
KC = 4
ASIZE = EMBED // KC


def _sc_gather(node_features, neighbors):
    B = neighbors.shape[0]
    D = node_features.shape[1]
    info = plsc.get_sparse_core_info()
    NW = info.num_cores * info.num_subcores
    b_per_w = B // NW
    CH = 200
    NB = 2
    n_ch = b_per_w // CH
    assert b_per_w % CH == 0
    n_super = n_ch // NB
    n_tail = n_ch - n_super * NB

    mesh = plsc.VectorSubcoreMesh(core_axis_name="c", subcore_axis_name="s")

    def body(nf_hbm, idx_hbm, out_hbm, idx_v, rows_v, gsem, ssem):
        cid = lax.axis_index("c")
        sid = lax.axis_index("s")
        wid = sid * info.num_cores + cid
        base = wid * b_per_w
        pltpu.sync_copy(idx_hbm.at[pl.ds(base, b_per_w)], idx_v)

        def super_step(io, carry):
            c0 = io * NB
            handles = [
                pltpu.async_copy(
                    nf_hbm.at[idx_v.at[pl.ds((c0 + b) * CH, CH)]],
                    rows_v.at[b],
                    gsem,
                )
                for b in range(NB)
            ]
            for b in range(NB):
                handles[b].wait()
                pltpu.async_copy(
                    rows_v.at[b],
                    out_hbm.at[pl.ds(base + (c0 + b) * CH, CH)],
                    ssem,
                ).wait()
            return carry

        lax.fori_loop(0, n_super, super_step, 0)
        for t in range(n_tail):
            c = n_super * NB + t
            pltpu.async_copy(
                nf_hbm.at[idx_v.at[pl.ds(c * CH, CH)]], rows_v.at[0], gsem
            ).wait()
            pltpu.async_copy(
                rows_v.at[0], out_hbm.at[pl.ds(base + c * CH, CH)], ssem
            ).wait()

    f = pl.kernel(
        body,
        out_type=jax.ShapeDtypeStruct((B, D), jnp.float32),
        mesh=mesh,
        scratch_types=[
            pltpu.VMEM((b_per_w,), jnp.int32),
            pltpu.VMEM((NB, CH, D), jnp.float32),
            pltpu.SemaphoreType.DMA,
            pltpu.SemaphoreType.DMA,
        ],
    )
    return f(node_features, neighbors)


def _routing(node_features, z_flat, num_node, num_neib):
    NBLK = 200
    RBLK = NBLK * num_neib
    z3d = z_flat.reshape(num_neib, num_node, EMBED)

    def kern(nf_ref, z_ref, out_ref):
        d_i = lax.broadcasted_iota(jnp.int32, (EMBED, EMBED), 0)
        d_j = lax.broadcasted_iota(jnp.int32, (EMBED, EMBED), 1)
        P = (d_i // ASIZE == d_j // ASIZE).astype(jnp.float32)
        Pall = jnp.full((EMBED, EMBED), 1.0 / ASIZE, dtype=jnp.float32)

        def seg_sums(x):
            return lax.dot_general(
                x, P, (((1,), (0,)), ((), ())),
                preferred_element_type=jnp.float32)

        def normalize(x):
            s2 = seg_sums(x * x)
            return x * jnp.minimum(lax.rsqrt(s2), 1e12)

        h = normalize(nf_ref[...])
        z = normalize(z_ref[...].reshape(RBLK, EMBED)).reshape(
            num_neib, NBLK, EMBED)

        u = normalize((1.0 / KC) * jnp.sum(z, axis=0) + h)
        for it in (1, 2):
            logits = seg_sums(
                (z * u[None, :, :]).reshape(RBLK, EMBED))
            e = jnp.exp(logits)
            denom = lax.dot_general(
                e, Pall, (((1,), (0,)), ((), ())),
                preferred_element_type=jnp.float32)
            w = (e / denom).reshape(num_neib, NBLK, EMBED)
            u = jnp.sum(z * w, axis=0) + h
            if it != 2:
                u = normalize(u)
        out_ref[...] = u

    num_blocks = num_node // NBLK
    return pl.pallas_call(
        kern,
        grid=(num_blocks,),
        in_specs=[
            pl.BlockSpec((NBLK, EMBED), lambda i: (i, 0)),
            pl.BlockSpec((num_neib, NBLK, EMBED), lambda i: (0, i, 0)),
        ],
        out_specs=pl.BlockSpec((NBLK, EMBED), lambda i: (i, 0)),
        out_shape=jax.ShapeDtypeStruct((num_node, EMBED), jnp.float32),
    )(node_features, z3d)


def kernel(node_features, neighbors, edgenode_iter):
    num_node = node_features.shape[0]
    num_neib = neighbors.shape[0] // num_node
    nb = neighbors.astype(jnp.int32)
    half = num_node // 2
    nb2 = nb.reshape(num_node, num_neib)
    nb0 = nb2[:half].T.reshape(-1)
    nb1 = nb2[half:].T.reshape(-1)
    z0 = _sc_gather(node_features, nb0)
    z1 = _sc_gather(node_features, nb1)
    u0 = _routing(node_features[:half], z0, half, num_neib)
    u1 = _routing(node_features[half:], z1, num_node - half, num_neib)
    return jnp.concatenate([u0, u1], axis=0)


# speedup vs baseline: 11.8242x; 1.0903x over previous
import jax
import jax.numpy as jnp
from jax import lax
from jax.experimental import pallas as pl
from jax.experimental.pallas import tpu as pltpu
from jax.experimental.pallas import tpu_sc as plsc

EMBED = 128
KC = 4
ASIZE = EMBED // KC


def _iota_P():
    d_i = lax.broadcasted_iota(jnp.int32, (EMBED, EMBED), 0)
    d_j = lax.broadcasted_iota(jnp.int32, (EMBED, EMBED), 1)
    return (d_i // ASIZE == d_j // ASIZE).astype(jnp.float32)


def _normalize(x, P):
    s2 = lax.dot_general(
        x * x, P, (((1,), (0,)), ((), ())), preferred_element_type=jnp.float32)
    return x * jnp.minimum(lax.rsqrt(s2), 1e12)


def _prep(node_features, num_node):
    BLK = 1000

    def kern(nf_ref, h32_ref):
        h32_ref[...] = _normalize(nf_ref[...], _iota_P())

    return pl.pallas_call(
        kern,
        grid=(num_node // BLK,),
        in_specs=[pl.BlockSpec((BLK, EMBED), lambda i: (i, 0))],
        out_specs=pl.BlockSpec((BLK, EMBED), lambda i: (i, 0)),
        out_shape=jax.ShapeDtypeStruct((num_node, EMBED), jnp.float32),
    )(node_features)


def _sc_gather(table, neighbors):
    B = neighbors.shape[0]
    D = table.shape[1]
    dt = table.dtype
    info = plsc.get_sparse_core_info()
    NW = info.num_cores * info.num_subcores
    b_per_w = B // NW
    CH = 200
    NB = 2
    n_ch = b_per_w // CH
    assert b_per_w % CH == 0
    n_super = n_ch // NB
    n_tail = n_ch - n_super * NB

    mesh = plsc.VectorSubcoreMesh(core_axis_name="c", subcore_axis_name="s")

    def body(nf_hbm, idx_hbm, out_hbm, idx_v, rows_v, gsem, ssem):
        cid = lax.axis_index("c")
        sid = lax.axis_index("s")
        wid = sid * info.num_cores + cid
        base = wid * b_per_w
        pltpu.sync_copy(idx_hbm.at[pl.ds(base, b_per_w)], idx_v)

        def super_step(io, carry):
            c0 = io * NB
            handles = [
                pltpu.async_copy(
                    nf_hbm.at[idx_v.at[pl.ds((c0 + b) * CH, CH)]],
                    rows_v.at[b],
                    gsem,
                )
                for b in range(NB)
            ]
            for b in range(NB):
                handles[b].wait()
                pltpu.async_copy(
                    rows_v.at[b],
                    out_hbm.at[pl.ds(base + (c0 + b) * CH, CH)],
                    ssem,
                ).wait()
            return carry

        lax.fori_loop(0, n_super, super_step, 0)
        for t in range(n_tail):
            c = n_super * NB + t
            pltpu.async_copy(
                nf_hbm.at[idx_v.at[pl.ds(c * CH, CH)]], rows_v.at[0], gsem
            ).wait()
            pltpu.async_copy(
                rows_v.at[0], out_hbm.at[pl.ds(base + c * CH, CH)], ssem
            ).wait()

    f = pl.kernel(
        body,
        out_type=jax.ShapeDtypeStruct((B, D), dt),
        mesh=mesh,
        scratch_types=[
            pltpu.VMEM((b_per_w,), jnp.int32),
            pltpu.VMEM((NB, CH, D), dt),
            pltpu.SemaphoreType.DMA,
            pltpu.SemaphoreType.DMA,
        ],
    )
    return f(table, neighbors)


def _routing(h32, z_flat, num_node, num_neib):
    NBLK = 200
    RBLK = NBLK * num_neib
    z3d = z_flat.reshape(num_neib, num_node, EMBED)

    def kern(h_ref, z_ref, out_ref):
        P = _iota_P()
        Pall = jnp.full((EMBED, EMBED), 1.0 / ASIZE, dtype=jnp.float32)

        def seg_sums(x):
            return lax.dot_general(
                x, P, (((1,), (0,)), ((), ())),
                preferred_element_type=jnp.float32)

        h = h_ref[...]
        z = z_ref[...]

        u = _normalize((1.0 / KC) * jnp.sum(z, axis=0) + h, P)
        for it in (1, 2):
            logits = seg_sums(
                (z * u[None, :, :]).reshape(RBLK, EMBED))
            e = jnp.exp(logits)
            denom = lax.dot_general(
                e, Pall, (((1,), (0,)), ((), ())),
                preferred_element_type=jnp.float32)
            w = (e / denom).reshape(num_neib, NBLK, EMBED)
            u = jnp.sum(z * w, axis=0) + h
            if it != 2:
                u = _normalize(u, P)
        out_ref[...] = u

    num_blocks = num_node // NBLK
    return pl.pallas_call(
        kern,
        grid=(num_blocks,),
        in_specs=[
            pl.BlockSpec((NBLK, EMBED), lambda i: (i, 0)),
            pl.BlockSpec((num_neib, NBLK, EMBED), lambda i: (0, i, 0)),
        ],
        out_specs=pl.BlockSpec((NBLK, EMBED), lambda i: (i, 0)),
        out_shape=jax.ShapeDtypeStruct((num_node, EMBED), jnp.float32),
    )(h32, z3d)


def kernel(node_features, neighbors, edgenode_iter):
    num_node = node_features.shape[0]
    num_neib = neighbors.shape[0] // num_node
    nb = neighbors.astype(jnp.int32)
    h32 = _prep(node_features, num_node)
    half = num_node // 2
    nb2 = nb.reshape(num_node, num_neib)
    nb0 = nb2[:half].T.reshape(-1)
    nb1 = nb2[half:].T.reshape(-1)
    z0 = _sc_gather(h32, nb0)
    z1 = _sc_gather(h32, nb1)
    u0 = _routing(h32[:half], z0, half, num_neib)
    u1 = _routing(h32[half:], z1, num_node - half, num_neib)
    return jnp.concatenate([u0, u1], axis=0)


# speedup vs baseline: 11.9762x; 1.0129x over previous
import jax
import jax.numpy as jnp
from jax import lax
from jax.experimental import pallas as pl
from jax.experimental.pallas import tpu as pltpu
from jax.experimental.pallas import tpu_sc as plsc

EMBED = 128
KC = 4
ASIZE = EMBED // KC


def _iota_P():
    d_i = lax.broadcasted_iota(jnp.int32, (EMBED, EMBED), 0)
    d_j = lax.broadcasted_iota(jnp.int32, (EMBED, EMBED), 1)
    return (d_i // ASIZE == d_j // ASIZE).astype(jnp.float32)


def _normalize(x, P):
    s2 = lax.dot_general(
        x * x, P, (((1,), (0,)), ((), ())), preferred_element_type=jnp.float32)
    return x * jnp.minimum(lax.rsqrt(s2), 1e12)


def _prep(node_features, num_node):
    BLK = 1000

    def kern(nf_ref, h32_ref):
        h32_ref[...] = _normalize(nf_ref[...], _iota_P())

    return pl.pallas_call(
        kern,
        grid=(num_node // BLK,),
        in_specs=[pl.BlockSpec((BLK, EMBED), lambda i: (i, 0))],
        out_specs=pl.BlockSpec((BLK, EMBED), lambda i: (i, 0)),
        out_shape=jax.ShapeDtypeStruct((num_node, EMBED), jnp.float32),
    )(node_features)


def _sc_gather(table, neighbors):
    B = neighbors.shape[0]
    D = table.shape[1]
    dt = table.dtype
    info = plsc.get_sparse_core_info()
    NW = info.num_cores * info.num_subcores
    b_per_w = B // NW
    CH = 400 if b_per_w % 400 == 0 else 200
    NB = 2
    n_ch = b_per_w // CH
    assert b_per_w % CH == 0
    n_super = n_ch // NB
    n_tail = n_ch - n_super * NB

    mesh = plsc.VectorSubcoreMesh(core_axis_name="c", subcore_axis_name="s")

    def body(nf_hbm, idx_hbm, out_hbm, idx_v, rows_v, gsem, ssem):
        cid = lax.axis_index("c")
        sid = lax.axis_index("s")
        wid = sid * info.num_cores + cid
        base = wid * b_per_w
        pltpu.sync_copy(idx_hbm.at[pl.ds(base, b_per_w)], idx_v)

        def super_step(io, carry):
            c0 = io * NB
            handles = [
                pltpu.async_copy(
                    nf_hbm.at[idx_v.at[pl.ds((c0 + b) * CH, CH)]],
                    rows_v.at[b],
                    gsem,
                )
                for b in range(NB)
            ]
            for b in range(NB):
                handles[b].wait()
                pltpu.async_copy(
                    rows_v.at[b],
                    out_hbm.at[pl.ds(base + (c0 + b) * CH, CH)],
                    ssem,
                ).wait()
            return carry

        lax.fori_loop(0, n_super, super_step, 0)
        for t in range(n_tail):
            c = n_super * NB + t
            pltpu.async_copy(
                nf_hbm.at[idx_v.at[pl.ds(c * CH, CH)]], rows_v.at[0], gsem
            ).wait()
            pltpu.async_copy(
                rows_v.at[0], out_hbm.at[pl.ds(base + c * CH, CH)], ssem
            ).wait()

    f = pl.kernel(
        body,
        out_type=jax.ShapeDtypeStruct((B, D), dt),
        mesh=mesh,
        scratch_types=[
            pltpu.VMEM((b_per_w,), jnp.int32),
            pltpu.VMEM((NB, CH, D), dt),
            pltpu.SemaphoreType.DMA,
            pltpu.SemaphoreType.DMA,
        ],
    )
    return f(table, neighbors)


def _routing(h32, z_flat, num_node, num_neib):
    NBLK = 200
    RBLK = NBLK * num_neib
    z3d = z_flat.reshape(num_neib, num_node, EMBED)

    def kern(h_ref, z_ref, out_ref):
        P = _iota_P()
        Pall = jnp.full((EMBED, EMBED), 1.0 / ASIZE, dtype=jnp.float32)

        def seg_sums(x):
            return lax.dot_general(
                x, P, (((1,), (0,)), ((), ())),
                preferred_element_type=jnp.float32)

        h = h_ref[...]
        z = z_ref[...]

        u = _normalize((1.0 / KC) * jnp.sum(z, axis=0) + h, P)
        for it in (1, 2):
            logits = seg_sums(
                (z * u[None, :, :]).reshape(RBLK, EMBED))
            e = jnp.exp(logits)
            denom = lax.dot_general(
                e, Pall, (((1,), (0,)), ((), ())),
                preferred_element_type=jnp.float32)
            w = (e / denom).reshape(num_neib, NBLK, EMBED)
            u = jnp.sum(z * w, axis=0) + h
            if it != 2:
                u = _normalize(u, P)
        out_ref[...] = u

    num_blocks = num_node // NBLK
    return pl.pallas_call(
        kern,
        grid=(num_blocks,),
        in_specs=[
            pl.BlockSpec((NBLK, EMBED), lambda i: (i, 0)),
            pl.BlockSpec((num_neib, NBLK, EMBED), lambda i: (0, i, 0)),
        ],
        out_specs=pl.BlockSpec((NBLK, EMBED), lambda i: (i, 0)),
        out_shape=jax.ShapeDtypeStruct((num_node, EMBED), jnp.float32),
    )(h32, z3d)


def kernel(node_features, neighbors, edgenode_iter):
    num_node = node_features.shape[0]
    num_neib = neighbors.shape[0] // num_node
    nb = neighbors.astype(jnp.int32)
    h32 = _prep(node_features, num_node)
    splits = (2000, 4000, 4000)
    nb2 = nb.reshape(num_node, num_neib)
    outs = []
    start = 0
    zs = []
    for sz in splits:
        nbs = nb2[start:start + sz].T.reshape(-1)
        zs.append(_sc_gather(h32, nbs))
        start += sz
    start = 0
    for sz, z in zip(splits, zs):
        outs.append(_routing(h32[start:start + sz], z, sz, num_neib))
        start += sz
    return jnp.concatenate(outs, axis=0)


# speedup vs baseline: 13.1306x; 1.0964x over previous
import jax
import jax.numpy as jnp
from jax import lax
from jax.experimental import pallas as pl
from jax.experimental.pallas import tpu as pltpu
from jax.experimental.pallas import tpu_sc as plsc

EMBED = 128
KC = 4
ASIZE = EMBED // KC


def _iota_P():
    d_i = lax.broadcasted_iota(jnp.int32, (EMBED, EMBED), 0)
    d_j = lax.broadcasted_iota(jnp.int32, (EMBED, EMBED), 1)
    return (d_i // ASIZE == d_j // ASIZE).astype(jnp.float32)


def _normalize(x, P):
    s2 = lax.dot_general(
        x * x, P, (((1,), (0,)), ((), ())), preferred_element_type=jnp.float32)
    return x * jnp.minimum(lax.rsqrt(s2), 1e12)


def _prep(node_features, num_node):
    BLK = 1000

    def kern(nf_ref, h32_ref):
        h32_ref[...] = _normalize(nf_ref[...], _iota_P())

    return pl.pallas_call(
        kern,
        grid=(num_node // BLK,),
        in_specs=[pl.BlockSpec((BLK, EMBED), lambda i: (i, 0))],
        out_specs=pl.BlockSpec((BLK, EMBED), lambda i: (i, 0)),
        out_shape=jax.ShapeDtypeStruct((num_node, EMBED), jnp.float32),
    )(node_features)


def _sc_gather(table, nbt_flat, sz):
    D = table.shape[1]
    dt = table.dtype
    info = plsc.get_sparse_core_info()
    NW = info.num_cores * info.num_subcores
    b_per_w = sz
    CH = 200
    NB = 2
    n_ch = b_per_w // CH
    assert b_per_w % CH == 0
    n_super = n_ch // NB
    n_tail = n_ch - n_super * NB

    mesh = plsc.VectorSubcoreMesh(core_axis_name="c", subcore_axis_name="s")

    def body(nf_hbm, idx_hbm, out_hbm, idx_v, rows_v, gsem, ssem):
        cid = lax.axis_index("c")
        sid = lax.axis_index("s")
        wid = sid * info.num_cores + cid
        base = wid * b_per_w
        pltpu.sync_copy(idx_hbm.at[pl.ds(base, b_per_w)], idx_v)

        def super_step(io, carry):
            c0 = io * NB
            bset = (io % 2) * NB
            @pl.when(io >= 2)
            def _():
                for b in range(NB):
                    pltpu.make_async_copy(
                        rows_v.at[bset + b],
                        out_hbm.at[pl.ds(base, CH)],
                        ssem,
                    ).wait()

            handles = [
                pltpu.async_copy(
                    nf_hbm.at[idx_v.at[pl.ds((c0 + b) * CH, CH)]],
                    rows_v.at[bset + b],
                    gsem,
                )
                for b in range(NB)
            ]
            for b in range(NB):
                handles[b].wait()
                pltpu.async_copy(
                    rows_v.at[bset + b],
                    out_hbm.at[pl.ds(base + (c0 + b) * CH, CH)],
                    ssem,
                )
            return carry

        lax.fori_loop(0, n_super, super_step, 0)
        n_out = min(n_super, 2) * NB
        for b in range(n_out):
            pltpu.make_async_copy(
                rows_v.at[b % (2 * NB)],
                out_hbm.at[pl.ds(base, CH)],
                ssem,
            ).wait()
        for t in range(n_tail):
            c = n_super * NB + t
            pltpu.async_copy(
                nf_hbm.at[idx_v.at[pl.ds(c * CH, CH)]], rows_v.at[0], gsem
            ).wait()
            pltpu.async_copy(
                rows_v.at[0], out_hbm.at[pl.ds(base + c * CH, CH)], ssem
            ).wait()

    f = pl.kernel(
        body,
        out_type=jax.ShapeDtypeStruct((sz * NW, D), dt),
        mesh=mesh,
        scratch_types=[
            pltpu.VMEM((b_per_w,), jnp.int32),
            pltpu.VMEM((2 * NB, CH, D), dt),
            pltpu.SemaphoreType.DMA,
            pltpu.SemaphoreType.DMA,
        ],
    )
    return f(table, nbt_flat)


def _routing(h32, z_flat, num_node, num_neib, normalize_z=False):
    NBLK = next(n for n in (800, 600, 400, 200) if num_node % n == 0)
    RBLK = NBLK * num_neib
    z3d = z_flat.reshape(num_neib, num_node, EMBED)

    def kern(h_ref, z_ref, out_ref):
        P = _iota_P()
        P16 = P.astype(jnp.bfloat16)
        Pall16 = jnp.full((EMBED, EMBED), 1.0 / ASIZE, dtype=jnp.bfloat16)

        def seg_sums16(x):
            return lax.dot_general(
                x, P16, (((1,), (0,)), ((), ())),
                preferred_element_type=jnp.float32)

        h = h_ref[...]
        z = z_ref[...]
        if normalize_z:
            z = _normalize(z.reshape(RBLK, EMBED), P).reshape(
                num_neib, NBLK, EMBED)
        z16 = z.astype(jnp.bfloat16)

        u = _normalize((1.0 / KC) * jnp.sum(z, axis=0) + h, P)
        for it in (1, 2):
            u16 = u.astype(jnp.bfloat16)
            logits = seg_sums16(
                (z16 * u16[None, :, :]).reshape(RBLK, EMBED))
            e = jnp.exp(logits)
            denom = lax.dot_general(
                e.astype(jnp.bfloat16), Pall16, (((1,), (0,)), ((), ())),
                preferred_element_type=jnp.float32)
            w = (e / denom).reshape(num_neib, NBLK, EMBED)
            u = jnp.sum(z * w, axis=0) + h
            if it != 2:
                u = _normalize(u, P)
        out_ref[...] = u

    num_blocks = num_node // NBLK
    return pl.pallas_call(
        kern,
        grid=(num_blocks,),
        in_specs=[
            pl.BlockSpec((NBLK, EMBED), lambda i: (i, 0)),
            pl.BlockSpec((num_neib, NBLK, EMBED), lambda i: (0, i, 0)),
        ],
        out_specs=pl.BlockSpec((NBLK, EMBED), lambda i: (i, 0)),
        out_shape=jax.ShapeDtypeStruct((num_node, EMBED), jnp.float32),
    )(h32, z3d)


def kernel(node_features, neighbors, edgenode_iter):
    num_node = node_features.shape[0]
    num_neib = neighbors.shape[0] // num_node
    nb = neighbors.astype(jnp.int32)
    h32 = _prep(node_features, num_node)
    splits = (2400, 3600, 4000)
    nb2 = nb.reshape(num_node, num_neib)
    outs = []
    start = 0
    zs = []
    for k, sz in enumerate(splits):
        nbs = nb2[start:start + sz].T.reshape(-1)
        zs.append(_sc_gather(node_features if k == 0 else h32, nbs, sz))
        start += sz
    start = 0
    for k, (sz, z) in enumerate(zip(splits, zs)):
        outs.append(_routing(h32[start:start + sz], z, sz, num_neib,
                             normalize_z=(k == 0)))
        start += sz
    return jnp.concatenate(outs, axis=0)
